# baseline (speedup 1.0000x reference)
import functools

import jax
import jax.numpy as jnp
from jax import lax
from jax.experimental import pallas as pl
from jax.experimental.pallas import tpu as pltpu
from jax.experimental.pallas import tpu_sc as plsc

_N = 10000
_E = 320000
_D = 128
_EPS = 1e-5

_NC = 2
_NS = 16
_NW = _NC * _NS
_CH = 128
_CPW = 80
_EPAD = _NW * _CPW * _CH
_NPAD = 10240
_RPT = _NPAD // _NS
_RB = 2
_BLK = 8
_CPW0 = 160


def _sc_body(with_cnt, h_hbm, src_hbm, dst_hbm, *refs):
    if with_cnt:
        acc_out, cnt_out = refs[0], refs[1]
        refs = refs[2:]
    else:
        acc_out = refs[0]
        refs = refs[1:]
    (src_v, dst_v, rows_v, ones_v, zc_v, acc_sh, cnt_sh) = refs[:7]
    gsems = refs[7:7 + _RB]
    isems = refs[7 + _RB:7 + _RB + 2]
    zsem = refs[7 + _RB + 2]

    c = lax.axis_index("c")
    s = lax.axis_index("s")
    base = s * _RPT
    cb0 = s * _CPW0

    def _buf(x):
        return rows_v.at[pl.ds(x * _CH, _CH)]

    def _irow(j):
        return ((j // _BLK) % 2) * _BLK + (j % _BLK)

    def _fire_i(cb, b):
        sl = pl.ds(cb + b * _BLK, _BLK)
        x = b % 2
        dst = pl.ds(x * _BLK, _BLK)
        pltpu.async_copy(src_hbm.at[sl], src_v.at[dst], isems[x])
        pltpu.async_copy(dst_hbm.at[sl], dst_v.at[dst], isems[x])

    def _drain_i(cb, b):
        sl = pl.ds(cb + b * _BLK, _BLK)
        x = b % 2
        dst = pl.ds(x * _BLK, _BLK)
        pltpu.make_async_copy(src_hbm.at[sl], src_v.at[dst],
                              isems[x]).wait()
        pltpu.make_async_copy(dst_hbm.at[sl], dst_v.at[dst],
                              isems[x]).wait()

    def _fire_g(j):
        pltpu.async_copy(h_hbm.at[src_v.at[_irow(j)]], _buf(j % _RB),
                         gsems[j % _RB])

    def _drain_g(j):
        pltpu.make_async_copy(h_hbm.at[src_v.at[_irow(j)]], _buf(j % _RB),
                              gsems[j % _RB]).wait()

    def _scatter(j):
        pltpu.sync_copy(_buf(j % _RB), acc_sh.at[dst_v.at[_irow(j)]],
                        add=True)
        if with_cnt:
            pltpu.sync_copy(ones_v, cnt_sh.at[dst_v.at[_irow(j)]],
                            add=True)

    def _prologue(cb):
        z16 = jnp.zeros((16,), jnp.float32)

        def _zrow(i, carry):
            for k in range(_D // 16):
                rows_v[i, pl.ds(k * 16, 16)] = z16
            return carry

        lax.fori_loop(0, _CH, _zrow, 0)
        for q in range(_RPT // _CH):
            pltpu.async_copy(rows_v.at[pl.ds(0, _CH)],
                             acc_sh.at[pl.ds(base + q * _CH, _CH)], zsem)

        if with_cnt:
            o16 = jnp.ones((16,), jnp.float32)
            for k in range(_CH // 16):
                ones_v[pl.ds(k * 16, 16)] = o16

            def _zc(i, carry):
                zc_v[pl.ds(i * 16, 16)] = z16
                return carry

            lax.fori_loop(0, _RPT // 16, _zc, 0)
            pltpu.async_copy(zc_v, cnt_sh.at[pl.ds(base, _RPT)], zsem)

        _fire_i(cb, 0)
        _fire_i(cb, 1)
        for q in range(_RPT // _CH):
            pltpu.make_async_copy(rows_v.at[pl.ds(0, _CH)],
                                  acc_sh.at[pl.ds(base + q * _CH, _CH)],
                                  zsem).wait()
        if with_cnt:
            pltpu.make_async_copy(zc_v, cnt_sh.at[pl.ds(base, _RPT)],
                                  zsem).wait()
        _drain_i(cb, 0)
        _fire_g(0)

    def _run(cb, cpw):
        nblk = cpw // _BLK
        for j in range(cpw):
            b = j // _BLK
            _drain_g(j)
            if j + 1 < cpw:
                if (j + 1) % _BLK == 0:
                    _drain_i(cb, b + 1)
                _fire_g(j + 1)
            _scatter(j)
            if j % _BLK == _BLK - 1 and b + 2 < nblk:
                _fire_i(cb, b + 2)

    @pl.when(c == 0)
    def _():
        _prologue(cb0)
        plsc.subcore_barrier()
        _run(cb0, _CPW0)
        plsc.subcore_barrier()
        pltpu.sync_copy(acc_sh.at[pl.ds(base, _RPT)],
                        acc_out.at[pl.ds(base, _RPT)])
        if with_cnt:
            pltpu.sync_copy(cnt_sh.at[pl.ds(base, _RPT)],
                            cnt_out.at[pl.ds(base, _RPT)])


@functools.lru_cache(maxsize=None)
def _make_sc(with_cnt):
    mesh = plsc.VectorSubcoreMesh(core_axis_name="c", subcore_axis_name="s",
                                  num_cores=_NC, num_subcores=_NS)
    out_type = [jax.ShapeDtypeStruct((_NPAD, _D), jnp.float32)]
    if with_cnt:
        out_type.append(jax.ShapeDtypeStruct((_NPAD,), jnp.float32))
    scratch = [
        pltpu.VMEM((2 * _BLK, _CH), jnp.int32),
        pltpu.VMEM((2 * _BLK, _CH), jnp.int32),
        pltpu.VMEM((_RB * _CH, _D), jnp.float32),
        pltpu.VMEM((_CH,), jnp.float32),
        pltpu.VMEM((_RPT,), jnp.float32),
        pltpu.VMEM_SHARED((_NPAD, _D), jnp.float32),
        pltpu.VMEM_SHARED((_NPAD,), jnp.float32),
    ] + [pltpu.SemaphoreType.DMA] * (_RB + 3)
    return pl.kernel(
        functools.partial(_sc_body, with_cnt),
        out_type=out_type,
        mesh=mesh,
        scratch_types=scratch,
    )


def _tc_body(final, x_ref, a_ref, c_ref, wl_ref, b_ref, wr_ref, g_ref,
             be_ref, o_ref):
    a = a_ref[...]
    cnt = c_ref[...]
    mean = a / jnp.maximum(cnt, 1.0)
    x = x_ref[...]
    o = lax.dot_general(mean, wl_ref[...], (((1,), (1,)), ((), ())),
                        preferred_element_type=jnp.float32)
    o = o + b_ref[...]
    o = o + lax.dot_general(x, wr_ref[...], (((1,), (1,)), ((), ())),
                            preferred_element_type=jnp.float32)
    if not final:
        scale = 1.0 / (1.0 + _EPS) ** 0.5
        o = o * (g_ref[...] * scale) + be_ref[...]
        o = jnp.maximum(o, 0.0) + x
    o_ref[...] = o


@functools.lru_cache(maxsize=None)
def _make_tc(final):
    grid = (_NPAD // _RPT,)
    in_specs = [
        pl.BlockSpec((_RPT, _D), lambda i: (i, 0)),
        pl.BlockSpec((_RPT, _D), lambda i: (i, 0)),
        pl.BlockSpec((_RPT, 1), lambda i: (i, 0)),
        pl.BlockSpec((_D, _D), lambda i: (0, 0)),
        pl.BlockSpec((1, _D), lambda i: (0, 0)),
        pl.BlockSpec((_D, _D), lambda i: (0, 0)),
        pl.BlockSpec((1, _D), lambda i: (0, 0)),
        pl.BlockSpec((1, _D), lambda i: (0, 0)),
    ]
    return pl.pallas_call(
        functools.partial(_tc_body, final),
        grid=grid,
        in_specs=in_specs,
        out_specs=pl.BlockSpec((_RPT, _D), lambda i: (i, 0)),
        out_shape=jax.ShapeDtypeStruct((_NPAD, _D), jnp.float32),
    )


def kernel(x, edge_index, Wl0, b0, Wr0, g0, be0, Wl1, b1, Wr1, g1, be1,
           Wl2, b2, Wr2):
    src = edge_index[0]
    dst = edge_index[1]
    npad = _EPAD - _E
    srcp = jnp.concatenate(
        [src, jnp.zeros((npad,), jnp.int32)]).reshape(_NW * _CPW, _CH)
    dstp = jnp.concatenate(
        [dst, _N + (jnp.arange(npad, dtype=jnp.int32) % (_NPAD - _N))]
    ).reshape(_NW * _CPW, _CH)
    xp = jnp.concatenate([x, jnp.zeros((_NPAD - _N, _D), jnp.float32)])

    sc0 = _make_sc(True)
    sc = _make_sc(False)
    tc = _make_tc(False)
    tc_final = _make_tc(True)

    ones2 = jnp.ones((1, _D), jnp.float32)
    zeros2 = jnp.zeros((1, _D), jnp.float32)

    A, cnt = sc0(xp, srcp, dstp)
    cntT = cnt.reshape(_NPAD, 1)
    h = tc(xp, A, cntT, Wl0, b0.reshape(1, _D), Wr0, g0.reshape(1, _D),
           be0.reshape(1, _D))
    (A,) = sc(h, srcp, dstp)
    h = tc(h, A, cntT, Wl1, b1.reshape(1, _D), Wr1, g1.reshape(1, _D),
           be1.reshape(1, _D))
    (A,) = sc(h, srcp, dstp)
    h = tc_final(h, A, cntT, Wl2, b2.reshape(1, _D), Wr2, ones2, zeros2)
    return h[:_N]


# speedup vs baseline: 1.0014x; 1.0014x over previous
import functools

import jax
import jax.numpy as jnp
from jax import lax
from jax.experimental import pallas as pl
from jax.experimental.pallas import tpu as pltpu
from jax.experimental.pallas import tpu_sc as plsc

_N = 10000
_E = 320000
_D = 128
_EPS = 1e-5

_NC = 2
_NS = 16
_NW = _NC * _NS
_CH = 128
_CPW = 80
_EPAD = _NW * _CPW * _CH
_NPAD = 10240
_RPT = _NPAD // _NS
_RB = 2
_BLK = 8
_CPW0 = 160


def _sc_body(with_cnt, h_hbm, src_hbm, dst_hbm, *refs):
    if with_cnt:
        acc_out, cnt_out = refs[0], refs[1]
        refs = refs[2:]
    else:
        acc_out = refs[0]
        refs = refs[1:]
    (src_v, dst_v, rows_v, ones_v, zc_v, acc_sh, cnt_sh) = refs[:7]
    gsems = refs[7:7 + _RB]
    isems = refs[7 + _RB:7 + _RB + 2]
    zsem = refs[7 + _RB + 2]

    c = lax.axis_index("c")
    s = lax.axis_index("s")
    base = s * _RPT
    cb0 = s * _CPW0

    def _buf(x):
        return rows_v.at[pl.ds(x * _CH, _CH)]

    def _irow(j):
        return ((j // _BLK) % 2) * _BLK + (j % _BLK)

    def _fire_i(cb, b, x):
        sl = pl.ds(cb + b * _BLK, _BLK)
        dst = pl.ds(x * _BLK, _BLK)
        pltpu.async_copy(src_hbm.at[sl], src_v.at[dst], isems[x])
        pltpu.async_copy(dst_hbm.at[sl], dst_v.at[dst], isems[x])

    def _drain_i(cb, b, x):
        sl = pl.ds(cb + b * _BLK, _BLK)
        dst = pl.ds(x * _BLK, _BLK)
        pltpu.make_async_copy(src_hbm.at[sl], src_v.at[dst],
                              isems[x]).wait()
        pltpu.make_async_copy(dst_hbm.at[sl], dst_v.at[dst],
                              isems[x]).wait()

    def _fire_g(k):
        pltpu.async_copy(h_hbm.at[src_v.at[_irow(k)]], _buf(k % _RB),
                         gsems[k % _RB])

    def _drain_g(k):
        pltpu.make_async_copy(h_hbm.at[src_v.at[_irow(k)]], _buf(k % _RB),
                              gsems[k % _RB]).wait()

    def _scatter(k):
        pltpu.sync_copy(_buf(k % _RB), acc_sh.at[dst_v.at[_irow(k)]],
                        add=True)
        if with_cnt:
            pltpu.sync_copy(ones_v, cnt_sh.at[dst_v.at[_irow(k)]],
                            add=True)

    def _prologue(cb):
        z16 = jnp.zeros((16,), jnp.float32)

        def _zrow(i, carry):
            for k in range(_D // 16):
                rows_v[i, pl.ds(k * 16, 16)] = z16
            return carry

        lax.fori_loop(0, _CH, _zrow, 0)
        for q in range(_RPT // _CH):
            pltpu.async_copy(rows_v.at[pl.ds(0, _CH)],
                             acc_sh.at[pl.ds(base + q * _CH, _CH)], zsem)

        if with_cnt:
            o16 = jnp.ones((16,), jnp.float32)
            for k in range(_CH // 16):
                ones_v[pl.ds(k * 16, 16)] = o16

            def _zc(i, carry):
                zc_v[pl.ds(i * 16, 16)] = z16
                return carry

            lax.fori_loop(0, _RPT // 16, _zc, 0)
            pltpu.async_copy(zc_v, cnt_sh.at[pl.ds(base, _RPT)], zsem)

        _fire_i(cb, 0, 0)
        _fire_i(cb, 1, 1)
        for q in range(_RPT // _CH):
            pltpu.make_async_copy(rows_v.at[pl.ds(0, _CH)],
                                  acc_sh.at[pl.ds(base + q * _CH, _CH)],
                                  zsem).wait()
        if with_cnt:
            pltpu.make_async_copy(zc_v, cnt_sh.at[pl.ds(base, _RPT)],
                                  zsem).wait()
        _drain_i(cb, 0, 0)
        _fire_g(0)

    def _run(cb, cpw):
        nblk = cpw // _BLK

        def _pair(t, carry):
            for k in range(2 * _BLK):
                j = t * (2 * _BLK) + k
                b2 = k // _BLK
                _drain_g(k)

                @pl.when(j + 1 < cpw)
                def _():
                    if (k + 1) % _BLK == 0:
                        _drain_i(cb, t * 2 + b2 + 1, (b2 + 1) % 2)
                    _fire_g(k + 1)

                _scatter(k)
                if k % _BLK == _BLK - 1:
                    @pl.when(t * 2 + b2 + 2 < nblk)
                    def _():
                        _fire_i(cb, t * 2 + b2 + 2, b2 % 2)
            return carry

        lax.fori_loop(0, cpw // (2 * _BLK), _pair, 0)

    @pl.when(c == 0)
    def _():
        _prologue(cb0)
        plsc.subcore_barrier()
        _run(cb0, _CPW0)
        plsc.subcore_barrier()
        pltpu.sync_copy(acc_sh.at[pl.ds(base, _RPT)],
                        acc_out.at[pl.ds(base, _RPT)])
        if with_cnt:
            pltpu.sync_copy(cnt_sh.at[pl.ds(base, _RPT)],
                            cnt_out.at[pl.ds(base, _RPT)])


@functools.lru_cache(maxsize=None)
def _make_sc(with_cnt):
    mesh = plsc.VectorSubcoreMesh(core_axis_name="c", subcore_axis_name="s",
                                  num_cores=_NC, num_subcores=_NS)
    out_type = [jax.ShapeDtypeStruct((_NPAD, _D), jnp.float32)]
    if with_cnt:
        out_type.append(jax.ShapeDtypeStruct((_NPAD,), jnp.float32))
    scratch = [
        pltpu.VMEM((2 * _BLK, _CH), jnp.int32),
        pltpu.VMEM((2 * _BLK, _CH), jnp.int32),
        pltpu.VMEM((_RB * _CH, _D), jnp.float32),
        pltpu.VMEM((_CH,), jnp.float32),
        pltpu.VMEM((_RPT,), jnp.float32),
        pltpu.VMEM_SHARED((_NPAD, _D), jnp.float32),
        pltpu.VMEM_SHARED((_NPAD,), jnp.float32),
    ] + [pltpu.SemaphoreType.DMA] * (_RB + 3)
    return pl.kernel(
        functools.partial(_sc_body, with_cnt),
        out_type=out_type,
        mesh=mesh,
        scratch_types=scratch,
    )


def _tc_body(final, x_ref, a_ref, c_ref, wl_ref, b_ref, wr_ref, g_ref,
             be_ref, o_ref):
    a = a_ref[...]
    cnt = c_ref[...]
    mean = a / jnp.maximum(cnt, 1.0)
    x = x_ref[...]
    o = lax.dot_general(mean, wl_ref[...], (((1,), (1,)), ((), ())),
                        preferred_element_type=jnp.float32)
    o = o + b_ref[...]
    o = o + lax.dot_general(x, wr_ref[...], (((1,), (1,)), ((), ())),
                            preferred_element_type=jnp.float32)
    if not final:
        scale = 1.0 / (1.0 + _EPS) ** 0.5
        o = o * (g_ref[...] * scale) + be_ref[...]
        o = jnp.maximum(o, 0.0) + x
    o_ref[...] = o


@functools.lru_cache(maxsize=None)
def _make_tc(final):
    grid = (_NPAD // _RPT,)
    in_specs = [
        pl.BlockSpec((_RPT, _D), lambda i: (i, 0)),
        pl.BlockSpec((_RPT, _D), lambda i: (i, 0)),
        pl.BlockSpec((_RPT, 1), lambda i: (i, 0)),
        pl.BlockSpec((_D, _D), lambda i: (0, 0)),
        pl.BlockSpec((1, _D), lambda i: (0, 0)),
        pl.BlockSpec((_D, _D), lambda i: (0, 0)),
        pl.BlockSpec((1, _D), lambda i: (0, 0)),
        pl.BlockSpec((1, _D), lambda i: (0, 0)),
    ]
    return pl.pallas_call(
        functools.partial(_tc_body, final),
        grid=grid,
        in_specs=in_specs,
        out_specs=pl.BlockSpec((_RPT, _D), lambda i: (i, 0)),
        out_shape=jax.ShapeDtypeStruct((_NPAD, _D), jnp.float32),
    )


def kernel(x, edge_index, Wl0, b0, Wr0, g0, be0, Wl1, b1, Wr1, g1, be1,
           Wl2, b2, Wr2):
    src = edge_index[0]
    dst = edge_index[1]
    npad = _EPAD - _E
    srcp = jnp.concatenate(
        [src, jnp.zeros((npad,), jnp.int32)]).reshape(_NW * _CPW, _CH)
    dstp = jnp.concatenate(
        [dst, _N + (jnp.arange(npad, dtype=jnp.int32) % (_NPAD - _N))]
    ).reshape(_NW * _CPW, _CH)
    xp = jnp.concatenate([x, jnp.zeros((_NPAD - _N, _D), jnp.float32)])

    sc0 = _make_sc(True)
    sc = _make_sc(False)
    tc = _make_tc(False)
    tc_final = _make_tc(True)

    ones2 = jnp.ones((1, _D), jnp.float32)
    zeros2 = jnp.zeros((1, _D), jnp.float32)

    A, cnt = sc0(xp, srcp, dstp)
    cntT = cnt.reshape(_NPAD, 1)
    h = tc(xp, A, cntT, Wl0, b0.reshape(1, _D), Wr0, g0.reshape(1, _D),
           be0.reshape(1, _D))
    (A,) = sc(h, srcp, dstp)
    h = tc(h, A, cntT, Wl1, b1.reshape(1, _D), Wr1, g1.reshape(1, _D),
           be1.reshape(1, _D))
    (A,) = sc(h, srcp, dstp)
    h = tc_final(h, A, cntT, Wl2, b2.reshape(1, _D), Wr2, ones2, zeros2)
    return h[:_N]


# speedup vs baseline: 3.9135x; 3.9080x over previous
import functools

import jax
import jax.numpy as jnp
from jax import lax
from jax.experimental import pallas as pl
from jax.experimental.pallas import tpu as pltpu
from jax.experimental.pallas import tpu_sc as plsc

_N = 10000
_E = 320000
_D = 128
_EPS = 1e-5

_NC = 2
_NS = 16
_NW = _NC * _NS
_CH = 128
_CPW = 80
_EPAD = _NW * _CPW * _CH
_NPAD = 10240
_RPT = _NPAD // _NS
_RB = 2
_BLK = 8
_NCHREAL = _E // _CH


def _sc_body(with_cnt, h_hbm, src_hbm, dst_hbm, *refs):
    if with_cnt:
        acc_out, cnt_out = refs[0], refs[1]
        refs = refs[2:]
    else:
        acc_out = refs[0]
        refs = refs[1:]
    (src_v, dst_v, rows_v, ones_v, zc_v, acc_sh, cnt_sh) = refs[:7]
    gsems = refs[7:7 + _RB]
    isems = refs[7 + _RB:7 + _RB + 2]
    zsem = refs[7 + _RB + 2]

    c = lax.axis_index("c")
    s = lax.axis_index("s")
    base = s * _RPT
    cb = (s * _NC + c) * _CPW

    def _buf(x):
        return rows_v.at[pl.ds(x * _CH, _CH)]

    def _irow(j):
        return ((j // _BLK) % 2) * _BLK + (j % _BLK)

    def _fire_i(cb, b, x):
        sl = pl.ds(cb + b * _BLK, _BLK)
        dst = pl.ds(x * _BLK, _BLK)
        pltpu.async_copy(src_hbm.at[sl], src_v.at[dst], isems[x])
        pltpu.async_copy(dst_hbm.at[sl], dst_v.at[dst], isems[x])

    def _drain_i(cb, b, x):
        sl = pl.ds(cb + b * _BLK, _BLK)
        dst = pl.ds(x * _BLK, _BLK)
        pltpu.make_async_copy(src_hbm.at[sl], src_v.at[dst],
                              isems[x]).wait()
        pltpu.make_async_copy(dst_hbm.at[sl], dst_v.at[dst],
                              isems[x]).wait()

    def _fire_g(k):
        pltpu.async_copy(h_hbm.at[src_v.at[_irow(k)]], _buf(k % _RB),
                         gsems[k % _RB])

    def _drain_g(k):
        pltpu.make_async_copy(h_hbm.at[src_v.at[_irow(k)]], _buf(k % _RB),
                              gsems[k % _RB]).wait()

    def _scatter(k, j):
        pltpu.sync_copy(_buf(k % _RB), acc_sh.at[dst_v.at[_irow(k)]],
                        add=True)
        if with_cnt:
            @pl.when(cb + j < _NCHREAL)
            def _():
                pltpu.sync_copy(ones_v, cnt_sh.at[dst_v.at[_irow(k)]],
                                add=True)

    def _prologue(cb):
        z16 = jnp.zeros((16,), jnp.float32)

        def _zrow(i, carry):
            for k in range(_D // 16):
                rows_v[i, pl.ds(k * 16, 16)] = z16
            return carry

        lax.fori_loop(0, _CH, _zrow, 0)
        for q in range(_RPT // _CH):
            pltpu.async_copy(rows_v.at[pl.ds(0, _CH)],
                             acc_sh.at[pl.ds(base + q * _CH, _CH)], zsem)

        if with_cnt:
            o16 = jnp.ones((16,), jnp.float32)
            for k in range(_CH // 16):
                ones_v[pl.ds(k * 16, 16)] = o16

            def _zc(i, carry):
                zc_v[pl.ds(i * 16, 16)] = z16
                return carry

            lax.fori_loop(0, _RPT // 16, _zc, 0)
            pltpu.async_copy(zc_v, cnt_sh.at[pl.ds(base, _RPT)], zsem)

        _fire_i(cb, 0, 0)
        _fire_i(cb, 1, 1)
        for q in range(_RPT // _CH):
            pltpu.make_async_copy(rows_v.at[pl.ds(0, _CH)],
                                  acc_sh.at[pl.ds(base + q * _CH, _CH)],
                                  zsem).wait()
        if with_cnt:
            pltpu.make_async_copy(zc_v, cnt_sh.at[pl.ds(base, _RPT)],
                                  zsem).wait()
        _drain_i(cb, 0, 0)
        _fire_g(0)

    def _run(cpw):
        nblk = cpw // _BLK

        def _pair(t, carry):
            for k in range(2 * _BLK):
                j = t * (2 * _BLK) + k
                b2 = k // _BLK
                _drain_g(k)

                @pl.when(j + 1 < cpw)
                def _():
                    if (k + 1) % _BLK == 0:
                        _drain_i(cb, t * 2 + b2 + 1, (b2 + 1) % 2)
                    _fire_g(k + 1)

                _scatter(k, j)
                if k % _BLK == _BLK - 1:
                    @pl.when(t * 2 + b2 + 2 < nblk)
                    def _():
                        _fire_i(cb, t * 2 + b2 + 2, b2 % 2)
            return carry

        lax.fori_loop(0, cpw // (2 * _BLK), _pair, 0)

    _prologue(cb)
    plsc.subcore_barrier()
    _run(_CPW)
    plsc.subcore_barrier()
    pltpu.sync_copy(acc_sh.at[pl.ds(base, _RPT)],
                    acc_out.at[c, pl.ds(base, _RPT)])
    if with_cnt:
        pltpu.sync_copy(cnt_sh.at[pl.ds(base, _RPT)],
                        cnt_out.at[c, pl.ds(base, _RPT)])


@functools.lru_cache(maxsize=None)
def _make_sc(with_cnt):
    mesh = plsc.VectorSubcoreMesh(core_axis_name="c", subcore_axis_name="s",
                                  num_cores=_NC, num_subcores=_NS)
    out_type = [jax.ShapeDtypeStruct((_NC, _NPAD, _D), jnp.float32)]
    if with_cnt:
        out_type.append(jax.ShapeDtypeStruct((_NC, _NPAD), jnp.float32))
    scratch = [
        pltpu.VMEM((2 * _BLK, _CH), jnp.int32),
        pltpu.VMEM((2 * _BLK, _CH), jnp.int32),
        pltpu.VMEM((_RB * _CH, _D), jnp.float32),
        pltpu.VMEM((_CH,), jnp.float32),
        pltpu.VMEM((_RPT,), jnp.float32),
        pltpu.VMEM_SHARED((_NPAD, _D), jnp.float32),
        pltpu.VMEM_SHARED((_NPAD,), jnp.float32),
    ] + [pltpu.SemaphoreType.DMA] * (_RB + 3)
    return pl.kernel(
        functools.partial(_sc_body, with_cnt),
        out_type=out_type,
        mesh=mesh,
        scratch_types=scratch,
    )


def _tc_body(final, x_ref, a_ref, c_ref, wl_ref, b_ref, wr_ref, g_ref,
             be_ref, o_ref):
    i = pl.program_id(0)
    a = a_ref[0] + a_ref[1]
    cnt = c_ref[:, 0:1] + c_ref[:, 1:2]
    mean = a / jnp.maximum(cnt, 1.0)
    x = x_ref[...]
    o = lax.dot_general(mean, wl_ref[...], (((1,), (1,)), ((), ())),
                        preferred_element_type=jnp.float32)
    o = o + b_ref[...]
    o = o + lax.dot_general(x, wr_ref[...], (((1,), (1,)), ((), ())),
                            preferred_element_type=jnp.float32)
    if not final:
        scale = 1.0 / (1.0 + _EPS) ** 0.5
        o = o * (g_ref[...] * scale) + be_ref[...]
        o = jnp.maximum(o, 0.0) + x
        row = i * _RPT + lax.broadcasted_iota(jnp.int32, o.shape, 0)
        o = jnp.where(row < _N, o, 0.0)
    o_ref[...] = o


@functools.lru_cache(maxsize=None)
def _make_tc(final):
    grid = (_NPAD // _RPT,)
    in_specs = [
        pl.BlockSpec((_RPT, _D), lambda i: (i, 0)),
        pl.BlockSpec((_NC, _RPT, _D), lambda i: (0, i, 0)),
        pl.BlockSpec((_RPT, _NC), lambda i: (i, 0)),
        pl.BlockSpec((_D, _D), lambda i: (0, 0)),
        pl.BlockSpec((1, _D), lambda i: (0, 0)),
        pl.BlockSpec((_D, _D), lambda i: (0, 0)),
        pl.BlockSpec((1, _D), lambda i: (0, 0)),
        pl.BlockSpec((1, _D), lambda i: (0, 0)),
    ]
    return pl.pallas_call(
        functools.partial(_tc_body, final),
        grid=grid,
        in_specs=in_specs,
        out_specs=pl.BlockSpec((_RPT, _D), lambda i: (i, 0)),
        out_shape=jax.ShapeDtypeStruct((_NPAD, _D), jnp.float32),
    )


def kernel(x, edge_index, Wl0, b0, Wr0, g0, be0, Wl1, b1, Wr1, g1, be1,
           Wl2, b2, Wr2):
    src = edge_index[0]
    dst = edge_index[1]
    npad = _EPAD - _E
    ar = jnp.arange(npad, dtype=jnp.int32)
    srcp = jnp.concatenate(
        [src, _N + ar % (_NPAD - _N)]).reshape(_NW * _CPW, _CH)
    dstp = jnp.concatenate([dst, ar % _N]).reshape(_NW * _CPW, _CH)
    xp = jnp.concatenate([x, jnp.zeros((_NPAD - _N, _D), jnp.float32)])

    sc0 = _make_sc(True)
    sc = _make_sc(False)
    tc = _make_tc(False)
    tc_final = _make_tc(True)

    ones2 = jnp.ones((1, _D), jnp.float32)
    zeros2 = jnp.zeros((1, _D), jnp.float32)

    A, cnt = sc0(xp, srcp, dstp)
    cntT = cnt.T
    h = tc(xp, A, cntT, Wl0, b0.reshape(1, _D), Wr0, g0.reshape(1, _D),
           be0.reshape(1, _D))
    (A,) = sc(h, srcp, dstp)
    h = tc(h, A, cntT, Wl1, b1.reshape(1, _D), Wr1, g1.reshape(1, _D),
           be1.reshape(1, _D))
    (A,) = sc(h, srcp, dstp)
    h = tc_final(h, A, cntT, Wl2, b2.reshape(1, _D), Wr2, ones2, zeros2)
    return h[:_N]
